# baseline (speedup 1.0000x reference)
import functools

import jax
import jax.numpy as jnp
from jax import lax
from jax.experimental import pallas as pl
from jax.experimental.pallas import tpu as pltpu
from jax.experimental.pallas import tpu_sc as plsc


def _x0_body(x_ref, wln_ref, bln_ref, x0_ref):
    x0_ref[...] = (
        jnp.dot(x_ref[...], wln_ref[...], preferred_element_type=jnp.float32)
        + bln_ref[...]
    )


def _h_body(x_ref, w_ref, h_ref):
    h_ref[...] = jnp.dot(x_ref[...], w_ref[...],
                         preferred_element_type=jnp.float32)


def _dense_linear(x, W_ln, b_ln, W1, W2):
    n, d_in = x.shape
    d_out = W_ln.shape[1]
    blk = 1000
    grid = n // blk
    x0 = pl.pallas_call(
        _x0_body,
        grid=(grid,),
        in_specs=[
            pl.BlockSpec((blk, d_in), lambda i: (i, 0)),
            pl.BlockSpec((d_in, d_out), lambda i: (0, 0)),
            pl.BlockSpec((1, d_out), lambda i: (0, 0)),
        ],
        out_specs=pl.BlockSpec((blk, d_out), lambda i: (i, 0)),
        out_shape=jax.ShapeDtypeStruct((n, d_out), jnp.float32),
    )(x, W_ln, b_ln.reshape(1, d_out))

    w_stack = jnp.concatenate([W1, W2], axis=0)
    h_cat = pl.pallas_call(
        _h_body,
        grid=(2 * grid,),
        in_specs=[
            pl.BlockSpec((blk, d_in), lambda j: (j % grid, 0)),
            pl.BlockSpec((d_in, d_out), lambda j: (j // grid, 0)),
        ],
        out_specs=pl.BlockSpec((blk, d_out), lambda j: (j, 0)),
        out_shape=jax.ShapeDtypeStruct((2 * n, d_out), jnp.float32),
    )(x, w_stack)
    return x0, h_cat


_NTILES = 16
_CH = 80
_NBUF = 3


def _make_sc_scatter(n, d, e):
    ept = e // _NTILES
    nch = ept // _CH
    rb = 80
    nblk = n // rb
    assert ept * _NTILES == e and nch * _CH == ept and nblk * rb == n
    assert d % 16 == 0 and nch >= _NBUF + 2
    nseg = d // 16

    mesh = plsc.VectorSubcoreMesh(core_axis_name="c", subcore_axis_name="s")

    vm = pltpu.VMEM
    @functools.partial(
        pl.kernel,
        mesh=mesh,
        out_type=(
            jax.ShapeDtypeStruct((n, d), jnp.float32),
            jax.ShapeDtypeStruct((n, d), jnp.float32),
        ),
        scratch_types=[
            [vm((2 * _CH,), jnp.int32)] * _NBUF,
            [vm((_CH,), jnp.float32)] * _NBUF,
            [vm((_CH,), jnp.int32)] * _NBUF,
            [vm((_CH, d), jnp.float32)] * _NBUF,
            vm((rb, d), jnp.float32),
            vm((d,), jnp.float32),
            pltpu.VMEM_SHARED((n, d), jnp.float32),
            [pltpu.SemaphoreType.DMA] * _NBUF,
            [pltpu.SemaphoreType.DMA] * _NBUF,
            [pltpu.SemaphoreType.DMA] * _NBUF,
        ],
    )
    def sc_scatter(h, ed, w, bc, out1, out2,
                   ebuf, w_v, dst_v, rows, bb, bv, acc,
                   sem_e, sem_g, sem_sc):
        c = lax.axis_index("c")
        s = lax.axis_index("s")
        eoff = ((c * _NTILES + s) * nch) * (2 * _CH)
        woff = ((c * _NTILES + s) * nch) * _CH
        hoff = c * n

        pltpu.sync_copy(bc.at[pl.ds(c * d, d)], bv)

        def fill_row(r, carry):
            for j in range(nseg):
                bb[r, pl.ds(j * 16, 16)] = bv[pl.ds(j * 16, 16)]
            return carry

        lax.fori_loop(0, rb, fill_row, 0)

        nblk_t = (nblk - 1 - s) // _NTILES + 1

        def init_body(k, carry):
            blk = (s + k * _NTILES) * rb
            pltpu.sync_copy(bb, acc.at[pl.ds(blk, rb)])
            return carry

        lax.fori_loop(0, nblk_t, init_body, 0)
        plsc.subcore_barrier()

        def fetch(i, b):
            pltpu.async_copy(ed.at[pl.ds(eoff + i * (2 * _CH), 2 * _CH)],
                             ebuf[b], sem_e[b])
            pltpu.async_copy(w.at[pl.ds(woff + i * _CH, _CH)],
                             w_v[b], sem_e[b])

        def wait_e(b):
            pltpu.make_async_copy(ed.at[pl.ds(0, 2 * _CH)], ebuf[b],
                                  sem_e[b]).wait()
            pltpu.make_async_copy(w.at[pl.ds(0, _CH)], w_v[b],
                                  sem_e[b]).wait()

        def gather(b):
            for g in range(_CH // 16):
                sl = pl.ds(g * 16, 16)
                ebuf[b][sl] = ebuf[b][sl] + hoff
                dst_v[b][sl] = ebuf[b][pl.ds(_CH + g * 16, 16)]
            pltpu.async_copy(h.at[ebuf[b].at[pl.ds(0, _CH)]], rows[b],
                             sem_g[b])

        def wait_g(b):
            pltpu.make_async_copy(h.at[ebuf[b].at[pl.ds(0, _CH)]], rows[b],
                                  sem_g[b]).wait()

        def scale(b):
            def group_body(g, carry):
                w16 = w_v[b][pl.ds(g * 16, 16)]
                for k in range(16):
                    wb = w16.at[jnp.full((16,), k, jnp.int32)].get(
                        mode="promise_in_bounds")
                    for j in range(nseg):
                        sl = pl.ds(j * 16, 16)
                        rows[b][g * 16 + k, sl] = rows[b][g * 16 + k, sl] * wb
                return carry

            lax.fori_loop(0, _CH // 16, group_body, 0)

        def scatter(b):
            pltpu.async_copy(rows[b], acc.at[dst_v[b]], sem_sc[b], add=True)

        def wait_sc(b):
            pltpu.make_async_copy(rows[b], acc.at[dst_v[b]], sem_sc[b]).wait()

        def body(i, b, do_wait_sc=True, do_fetch=True, do_gather=True):
            nb = (b + 2) % _NBUF
            if do_wait_sc:
                wait_sc(nb)
            if do_fetch:
                fetch(i + 2, nb)
            wait_g(b)
            scale(b)
            if do_gather:
                wait_e((b + 1) % _NBUF)
                gather((b + 1) % _NBUF)
            scatter(b)

        fetch(0, 0)
        fetch(1, 1)
        wait_e(0)
        gather(0)
        body(0, 0, do_wait_sc=False)

        nq = (nch - 3) // _NBUF
        def loop_body(q, carry):
            for idx, b in enumerate((1, 2, 0)):
                body(1 + q * _NBUF + idx, b)
            return carry

        lax.fori_loop(0, nq, loop_body, 0)
        for i in range(1 + nq * _NBUF, nch):
            body(i, i % _NBUF,
                 do_fetch=(i + 2 < nch), do_gather=(i + 1 < nch))
        wait_sc((nch - 1) % _NBUF)

        plsc.subcore_barrier()

        def make_out_body(out):
            def out_body(k, carry):
                blk = (s + k * _NTILES) * rb
                pltpu.sync_copy(acc.at[pl.ds(blk, rb)], out.at[pl.ds(blk, rb)])
                return carry
            return out_body

        @pl.when(c == 0)
        def _():
            lax.fori_loop(0, nblk_t, make_out_body(out1), 0)

        @pl.when(c == 1)
        def _():
            lax.fori_loop(0, nblk_t, make_out_body(out2), 0)

    return sc_scatter


@jax.jit
def kernel(x, edge_index, edge_weight, edge_index2, edge_weight2,
           W_ln, b_ln, W1, b1, W2, b2):
    n, _ = x.shape
    d = W_ln.shape[1]
    e = edge_weight.shape[0]
    x0, h_cat = _dense_linear(x, W_ln, b_ln, W1, W2)

    def pack(ei_b):
        return ei_b.reshape(2, _NTILES, -1, _CH).transpose(1, 2, 0, 3)

    ed = jnp.stack([pack(edge_index), pack(edge_index2)]).reshape(-1)
    w_cat = jnp.concatenate(
        [edge_weight, edge_weight2]).reshape(2, _NTILES, -1, _CH).reshape(-1)
    b_cat = jnp.concatenate([b1, b2])
    sc = _make_sc_scatter(n, d, e)
    x1, x2 = sc(h_cat, ed, w_cat, b_cat)
    return (x0, x1, x2)


# speedup vs baseline: 1.5057x; 1.5057x over previous
import functools

import jax
import jax.numpy as jnp
from jax import lax
from jax.experimental import pallas as pl
from jax.experimental.pallas import tpu as pltpu
from jax.experimental.pallas import tpu_sc as plsc


def _x0_body(x_ref, wln_ref, bln_ref, x0_ref):
    x0_ref[...] = (
        jnp.dot(x_ref[...], wln_ref[...], preferred_element_type=jnp.float32)
        + bln_ref[...]
    )


def _h_body(x_ref, w_ref, h_ref):
    h_ref[...] = jnp.dot(x_ref[...], w_ref[...],
                         preferred_element_type=jnp.float32)


def _dense_linear(x, W_ln, b_ln, W1, W2):
    n, d_in = x.shape
    d_out = W_ln.shape[1]
    blk = 1000
    grid = n // blk
    x0 = pl.pallas_call(
        _x0_body,
        grid=(grid,),
        in_specs=[
            pl.BlockSpec((blk, d_in), lambda i: (i, 0)),
            pl.BlockSpec((d_in, d_out), lambda i: (0, 0)),
            pl.BlockSpec((1, d_out), lambda i: (0, 0)),
        ],
        out_specs=pl.BlockSpec((blk, d_out), lambda i: (i, 0)),
        out_shape=jax.ShapeDtypeStruct((n, d_out), jnp.float32),
    )(x, W_ln, b_ln.reshape(1, d_out))

    w_stack = jnp.concatenate([W1, W2], axis=0)
    h_cat = pl.pallas_call(
        _h_body,
        grid=(2 * grid,),
        in_specs=[
            pl.BlockSpec((blk, d_in), lambda j: (j % grid, 0)),
            pl.BlockSpec((d_in, d_out), lambda j: (j // grid, 0)),
        ],
        out_specs=pl.BlockSpec((blk, d_out), lambda j: (j, 0)),
        out_shape=jax.ShapeDtypeStruct((2 * n, d_out), jnp.float32),
    )(x, w_stack)
    return x0, h_cat


_NTILES = 16
_CH = 80
_NBUF = 3


def _make_sc_scatter(n, d, e):
    ept = e // _NTILES
    nch = ept // _CH
    rb = 80
    nblk = n // rb
    assert ept * _NTILES == e and nch * _CH == ept and nblk * rb == n
    assert d % 16 == 0 and nch >= _NBUF + 2
    nseg = d // 16

    mesh = plsc.VectorSubcoreMesh(core_axis_name="c", subcore_axis_name="s")

    vm = pltpu.VMEM
    @functools.partial(
        pl.kernel,
        mesh=mesh,
        out_type=(
            jax.ShapeDtypeStruct((n, d), jnp.float32),
            jax.ShapeDtypeStruct((n, d), jnp.float32),
        ),
        scratch_types=[
            [vm((2 * _CH,), jnp.int32)] * _NBUF,
            [vm((_CH,), jnp.float32)] * _NBUF,
            [vm((_CH,), jnp.int32)] * _NBUF,
            [vm((_CH, d), jnp.float32)] * _NBUF,
            vm((rb, d), jnp.float32),
            vm((d,), jnp.float32),
            pltpu.VMEM_SHARED((n, d), jnp.float32),
            [pltpu.SemaphoreType.DMA] * _NBUF,
            [pltpu.SemaphoreType.DMA] * _NBUF,
            [pltpu.SemaphoreType.DMA] * _NBUF,
        ],
    )
    def sc_scatter(h, ed, w, bc, out1, out2,
                   ebuf, w_v, dst_v, rows, bb, bv, acc,
                   sem_e, sem_g, sem_sc):
        c = lax.axis_index("c")
        s = lax.axis_index("s")
        eoff = ((c * _NTILES + s) * nch) * (2 * _CH)
        woff = ((c * _NTILES + s) * nch) * _CH
        hoff = c * n

        pltpu.sync_copy(bc.at[pl.ds(c * d, d)], bv)

        def fill_row(r, carry):
            for j in range(nseg):
                bb[r, pl.ds(j * 16, 16)] = bv[pl.ds(j * 16, 16)]
            return carry

        lax.fori_loop(0, rb, fill_row, 0)

        nblk_t = (nblk - 1 - s) // _NTILES + 1

        def init_body(k, carry):
            blk = (s + k * _NTILES) * rb
            pltpu.sync_copy(bb, acc.at[pl.ds(blk, rb)])
            return carry

        lax.fori_loop(0, nblk_t, init_body, 0)
        plsc.subcore_barrier()

        def fetch(i, b):
            pltpu.async_copy(ed.at[pl.ds(eoff + i * (2 * _CH), 2 * _CH)],
                             ebuf[b], sem_e[b])
            pltpu.async_copy(w.at[pl.ds(woff + i * _CH, _CH)],
                             w_v[b], sem_e[b])

        def wait_e(b):
            pltpu.make_async_copy(ed.at[pl.ds(0, 2 * _CH)], ebuf[b],
                                  sem_e[b]).wait()
            pltpu.make_async_copy(w.at[pl.ds(0, _CH)], w_v[b],
                                  sem_e[b]).wait()

        def gather(b):
            for g in range(_CH // 16):
                sl = pl.ds(g * 16, 16)
                ebuf[b][sl] = ebuf[b][sl] + hoff
                dst_v[b][sl] = ebuf[b][pl.ds(_CH + g * 16, 16)]
            pltpu.async_copy(h.at[ebuf[b].at[pl.ds(0, _CH)]], rows[b],
                             sem_g[b])

        def wait_g(b):
            pltpu.make_async_copy(h.at[ebuf[b].at[pl.ds(0, _CH)]], rows[b],
                                  sem_g[b]).wait()

        def scale(b):
            def group_body(g, carry):
                w16 = w_v[b][pl.ds(g * 16, 16)]
                for k in range(16):
                    wb = w16.at[jnp.full((16,), k, jnp.int32)].get(
                        mode="promise_in_bounds")
                    for j in range(nseg):
                        sl = pl.ds(j * 16, 16)
                        rows[b][g * 16 + k, sl] = rows[b][g * 16 + k, sl] * wb
                return carry

            lax.fori_loop(0, _CH // 16, group_body, 0)

        def scatter(b):
            pltpu.async_copy(rows[b], acc.at[dst_v[b]], sem_sc[b], add=True)

        def wait_sc(b):
            pltpu.make_async_copy(rows[b], acc.at[dst_v[b]], sem_sc[b]).wait()

        def body(i, b, do_wait_sc=True, do_fetch=True, do_gather=True):
            nb = (b + 2) % _NBUF
            if do_fetch:
                fetch(i + 2, nb)
            if do_gather:
                wait_e((b + 1) % _NBUF)
                if do_wait_sc:
                    wait_sc((b + 1) % _NBUF)
                gather((b + 1) % _NBUF)
            wait_g(b)
            scale(b)
            scatter(b)

        fetch(0, 0)
        fetch(1, 1)
        wait_e(0)
        gather(0)
        body(0, 0, do_wait_sc=False)
        body(1, 1, do_wait_sc=False)

        nq = (nch - 4) // _NBUF
        def loop_body(q, carry):
            for idx, b in enumerate((2, 0, 1)):
                body(2 + q * _NBUF + idx, b)
            return carry

        lax.fori_loop(0, nq, loop_body, 0)
        for i in range(2 + nq * _NBUF, nch):
            body(i, i % _NBUF,
                 do_fetch=(i + 2 < nch), do_gather=(i + 1 < nch))
        for j in (nch - 3, nch - 2, nch - 1):
            wait_sc(j % _NBUF)

        plsc.subcore_barrier()

        def make_out_body(out):
            def out_body(k, carry):
                blk = (s + k * _NTILES) * rb
                pltpu.sync_copy(acc.at[pl.ds(blk, rb)], out.at[pl.ds(blk, rb)])
                return carry
            return out_body

        @pl.when(c == 0)
        def _():
            lax.fori_loop(0, nblk_t, make_out_body(out1), 0)

        @pl.when(c == 1)
        def _():
            lax.fori_loop(0, nblk_t, make_out_body(out2), 0)

    return sc_scatter


@jax.jit
def kernel(x, edge_index, edge_weight, edge_index2, edge_weight2,
           W_ln, b_ln, W1, b1, W2, b2):
    n, _ = x.shape
    d = W_ln.shape[1]
    e = edge_weight.shape[0]
    x0, h_cat = _dense_linear(x, W_ln, b_ln, W1, W2)

    def pack(ei_b):
        return ei_b.reshape(2, _NTILES, -1, _CH).transpose(1, 2, 0, 3)

    ed = jnp.stack([pack(edge_index), pack(edge_index2)]).reshape(-1)
    w_cat = jnp.concatenate(
        [edge_weight, edge_weight2]).reshape(2, _NTILES, -1, _CH).reshape(-1)
    b_cat = jnp.concatenate([b1, b2])
    sc = _make_sc_scatter(n, d, e)
    x1, x2 = sc(h_cat, ed, w_cat, b_cat)
    return (x0, x1, x2)
